# baseline (speedup 1.0000x reference)
import dataclasses
import functools

import jax
import jax.numpy as jnp
from jax import lax
from jax.experimental import pallas as pl
from jax.experimental.pallas import tpu as pltpu
from jax.experimental.pallas import tpu_sc as plsc

_NUM_BINS = 65536
_N = 16777216
_NC = 2
_NS = 16
_L = 16
_NW = _NC * _NS
_PER_W = _N // _NW
_CHUNK = 8192
_NCHUNK = _PER_W // _CHUNK

_mesh = plsc.VectorSubcoreMesh(core_axis_name="c", subcore_axis_name="s")

_sc_params = pltpu.CompilerParams()
if "needs_layout_passes" in pltpu.CompilerParams.__dataclass_fields__:
    _sc_params = dataclasses.replace(_sc_params, needs_layout_passes=False)


@functools.partial(
    pl.kernel,
    out_type=jax.ShapeDtypeStruct((_NW, _NUM_BINS), jnp.int32),
    mesh=_mesh,
    scratch_types=[
        pltpu.VMEM((_NUM_BINS,), jnp.int32),
        pltpu.VMEM((_CHUNK,), jnp.int32),
        pltpu.VMEM((_CHUNK,), jnp.int32),
        pltpu.SemaphoreType.DMA,
        pltpu.SemaphoreType.DMA,
    ],
    compiler_params=_sc_params,
)
def _sc_hist(inp_hbm, out_hbm, hist, buf_a, buf_b, sem_a, sem_b):
    wid = lax.axis_index("s") * _NC + lax.axis_index("c")
    base = wid * _PER_W

    zeros = jnp.zeros((_L,), jnp.int32)
    ones = jnp.ones((_L,), jnp.int32)

    def start(g, buf, sem):
        pltpu.async_copy(inp_hbm.at[pl.ds(base + g * _CHUNK, _CHUNK)], buf, sem)

    def wait(buf, sem):
        pltpu.make_async_copy(inp_hbm.at[pl.ds(base, _CHUNK)], buf, sem).wait()

    def process(buf):
        @plsc.parallel_loop(0, _CHUNK, step=_L, unroll=16)
        def _upd(i):
            idx = buf[pl.ds(i, _L)]
            plsc.addupdate_scatter(hist, [idx], ones)

    start(0, buf_a, sem_a)
    start(1, buf_b, sem_b)

    @pl.loop(0, _NUM_BINS, step=_L * 32)
    def _zero(i):
        for j in range(32):
            hist[pl.ds(i + j * _L, _L)] = zeros

    @pl.loop(0, _NCHUNK, step=2)
    def _chunks(g):
        wait(buf_a, sem_a)
        process(buf_a)

        @pl.when(g + 2 < _NCHUNK)
        def _():
            start(g + 2, buf_a, sem_a)

        wait(buf_b, sem_b)
        process(buf_b)

        @pl.when(g + 3 < _NCHUNK)
        def _():
            start(g + 3, buf_b, sem_b)

    pltpu.sync_copy(hist, out_hbm.at[wid])


_RCOLS = 32768


def _reduce_body(x_ref, o_ref):
    x = x_ref[...]
    s = x[0:8] + x[8:16] + x[16:24] + x[24:32]
    o_ref[...] = jnp.sum(s, axis=0)


_tc_reduce = pl.pallas_call(
    _reduce_body,
    out_shape=jax.ShapeDtypeStruct((_NUM_BINS,), jnp.int32),
    in_specs=[pl.BlockSpec((_NW, _RCOLS), lambda i: (0, i))],
    out_specs=pl.BlockSpec((_RCOLS,), lambda i: (i,)),
    grid=(_NUM_BINS // _RCOLS,),
)


def kernel(input):
    partials = _sc_hist(input)
    return _tc_reduce(partials)


# speedup vs baseline: 1.1469x; 1.1469x over previous
import dataclasses
import functools

import jax
import jax.numpy as jnp
from jax import lax
from jax.experimental import pallas as pl
from jax.experimental.pallas import tpu as pltpu
from jax.experimental.pallas import tpu_sc as plsc

_NUM_BINS = 65536
_N = 16777216
_NC = 2
_NS = 16
_L = 16
_NW = _NC * _NS
_PER_W = _N // _NW
_CHUNK = 16384
_NCHUNK = _PER_W // _CHUNK

_mesh = plsc.VectorSubcoreMesh(core_axis_name="c", subcore_axis_name="s")

_sc_params = pltpu.CompilerParams()
if "needs_layout_passes" in pltpu.CompilerParams.__dataclass_fields__:
    _sc_params = dataclasses.replace(_sc_params, needs_layout_passes=False)


@functools.partial(
    pl.kernel,
    out_type=jax.ShapeDtypeStruct((_NW, _NUM_BINS), jnp.int32),
    mesh=_mesh,
    scratch_types=[
        pltpu.VMEM((_NUM_BINS,), jnp.int32),
        pltpu.VMEM((_CHUNK,), jnp.int32),
        pltpu.VMEM((_CHUNK,), jnp.int32),
        pltpu.VMEM((_CHUNK,), jnp.int32),
        pltpu.SemaphoreType.DMA,
        pltpu.SemaphoreType.DMA,
        pltpu.SemaphoreType.DMA,
    ],
    compiler_params=_sc_params,
)
def _sc_hist(inp_hbm, out_hbm, hist, buf_a, buf_b, buf_c, sem_a, sem_b, sem_c):
    wid = lax.axis_index("s") * _NC + lax.axis_index("c")
    base = wid * _PER_W

    zeros = jnp.zeros((_L,), jnp.int32)
    ones = jnp.ones((_L,), jnp.int32)

    def start(g, buf, sem):
        pltpu.async_copy(inp_hbm.at[pl.ds(base + g * _CHUNK, _CHUNK)], buf, sem)

    def wait(buf, sem):
        pltpu.make_async_copy(inp_hbm.at[pl.ds(base, _CHUNK)], buf, sem).wait()

    def process(buf):
        @plsc.parallel_loop(0, _CHUNK, step=_L, unroll=16)
        def _upd(i):
            idx = buf[pl.ds(i, _L)]
            plsc.addupdate_scatter(hist, [idx], ones)

    start(0, buf_a, sem_a)
    start(1, buf_b, sem_b)
    start(2, buf_c, sem_c)

    @pl.loop(0, _NUM_BINS, step=_L * 32)
    def _zero(i):
        for j in range(32):
            hist[pl.ds(i + j * _L, _L)] = zeros

    @pl.loop(0, _NCHUNK, step=3)
    def _chunks(g):
        for k, (buf, sem) in enumerate(
            ((buf_a, sem_a), (buf_b, sem_b), (buf_c, sem_c))
        ):
            @pl.when(g + k < _NCHUNK)
            def _():
                wait(buf, sem)
                process(buf)

                @pl.when(g + k + 3 < _NCHUNK)
                def _():
                    start(g + k + 3, buf, sem)

    pltpu.sync_copy(hist, out_hbm.at[wid])


_RCOLS = 32768


def _reduce_body(x_ref, o_ref):
    x = x_ref[...]
    s = x[0:8] + x[8:16] + x[16:24] + x[24:32]
    o_ref[...] = jnp.sum(s, axis=0)


_tc_reduce = pl.pallas_call(
    _reduce_body,
    out_shape=jax.ShapeDtypeStruct((_NUM_BINS,), jnp.int32),
    in_specs=[pl.BlockSpec((_NW, _RCOLS), lambda i: (0, i))],
    out_specs=pl.BlockSpec((_RCOLS,), lambda i: (i,)),
    grid=(_NUM_BINS // _RCOLS,),
)


def kernel(input):
    partials = _sc_hist(input)
    return _tc_reduce(partials)
